# Initial kernel scaffold; baseline (speedup 1.0000x reference)
import jax
import jax.numpy as jnp
from jax.experimental import pallas as pl


def kernel(x, edge_index, u, w, W1, b1, W2, b2, W3, b3, W4, b4, W5, b5, Wlin, blin):
    raise NotImplementedError("write your pallas kernel here")


# speedup vs baseline: 8.9408x; 8.9408x over previous
import functools

import jax
import jax.numpy as jnp
from jax import lax
from jax.experimental import pallas as pl
from jax.experimental.pallas import tpu as pltpu
from jax.experimental.pallas import tpu_sc as plsc

NC = 2
NS = 16
B = 80
DEGW = 16
NEG_SLOPE = 0.01


def _sc_mesh():
    return plsc.VectorSubcoreMesh(core_axis_name="c", subcore_axis_name="s")


_SC_PARAMS = pltpu.CompilerParams(use_tc_tiling_on_sc=False)


def _degree_partials(dst2d, zeros16, ones16, n):
    nchunk, b = dst2d.shape
    nb = nchunk // (NC * NS)
    rpt = n // NS

    @functools.partial(
        pl.kernel,
        out_type=jax.ShapeDtypeStruct((NC, n, DEGW), jnp.float32),
        mesh=_sc_mesh(),
        compiler_params=_SC_PARAMS,
        scratch_types=[
            pltpu.VMEM((b,), jnp.int32),
            pltpu.VMEM((b, DEGW), jnp.float32),
            pltpu.VMEM_SHARED((n, DEGW), jnp.float32),
        ],
    )
    def deg_kernel(dst_hbm, zeros_hbm, ones_hbm, out_hbm, dst_v, ones_v, acc_sh):
        cid = lax.axis_index("c")
        sid = lax.axis_index("s")
        r0 = sid * rpt
        pltpu.sync_copy(zeros_hbm.at[pl.ds(r0, rpt)], acc_sh.at[pl.ds(r0, rpt)])
        pltpu.sync_copy(ones_hbm, ones_v)
        plsc.subcore_barrier()
        wid = cid * NS + sid

        def body(k, carry):
            pltpu.sync_copy(dst_hbm.at[wid * nb + k], dst_v)
            pltpu.sync_copy(ones_v, acc_sh.at[dst_v], add=True)
            return carry

        lax.fori_loop(0, nb, body, 0)
        plsc.subcore_barrier()
        pltpu.sync_copy(acc_sh.at[pl.ds(r0, rpt)], out_hbm.at[cid, pl.ds(r0, rpt)])

    return deg_kernel(dst2d, zeros16, ones16)


def _sc_spmm(hp, zeros, src2d, dst2d, n, d):
    nchunk, b = src2d.shape
    nb = nchunk // (NC * NS)
    rpt = n // NS

    @functools.partial(
        pl.kernel,
        out_type=jax.ShapeDtypeStruct((NC, n, d), jnp.float32),
        mesh=_sc_mesh(),
        compiler_params=_SC_PARAMS,
        scratch_types=[
            pltpu.VMEM((b,), jnp.int32),
            pltpu.VMEM((b,), jnp.int32),
            pltpu.VMEM((b, d), jnp.float32),
            pltpu.VMEM_SHARED((n, d), jnp.float32),
            pltpu.SemaphoreType.DMA,
        ],
    )
    def spmm_kernel(hp_hbm, zeros_hbm, src_hbm, dst_hbm, out_hbm,
                    src_v, dst_v, rows_v, acc_sh, sem):
        cid = lax.axis_index("c")
        sid = lax.axis_index("s")
        r0 = sid * rpt

        @pl.when(cid == 0)
        def _():
            pltpu.sync_copy(hp_hbm.at[pl.ds(r0, rpt)], acc_sh.at[pl.ds(r0, rpt)])

        @pl.when(cid != 0)
        def _():
            pltpu.sync_copy(zeros_hbm.at[pl.ds(r0, rpt)], acc_sh.at[pl.ds(r0, rpt)])

        plsc.subcore_barrier()
        wid = cid * NS + sid

        def body(k, carry):
            chunk = wid * nb + k
            pltpu.sync_copy(src_hbm.at[chunk], src_v)
            pltpu.sync_copy(dst_hbm.at[chunk], dst_v)
            pltpu.async_copy(hp_hbm.at[src_v], rows_v, sem).wait()
            pltpu.sync_copy(rows_v, acc_sh.at[dst_v], add=True)
            return carry

        lax.fori_loop(0, nb, body, 0)
        plsc.subcore_barrier()
        pltpu.sync_copy(acc_sh.at[pl.ds(r0, rpt)], out_hbm.at[cid, pl.ds(r0, rpt)])

    return spmm_kernel(hp, zeros, src2d, dst2d)


def _tc_first(x, w1x, c_row, deg0, deg1, br=1000):
    n, d = x.shape
    hd = w1x.shape[1]
    grid = n // br

    def body(x_ref, w_ref, c_ref, d0_ref, d1_ref, hp_ref, dinv_ref):
        deg = d0_ref[:, 0:1] + d1_ref[:, 0:1] + 1.0
        dinv = lax.rsqrt(deg)
        h = jnp.dot(x_ref[...], w_ref[...],
                    preferred_element_type=jnp.float32) + c_ref[...]
        hp_ref[...] = h * dinv
        dinv_ref[...] = dinv

    return pl.pallas_call(
        body,
        grid=(grid,),
        in_specs=[
            pl.BlockSpec((br, d), lambda i: (i, 0)),
            pl.BlockSpec((d, hd), lambda i: (0, 0)),
            pl.BlockSpec((1, hd), lambda i: (0, 0)),
            pl.BlockSpec((br, DEGW), lambda i: (i, 0)),
            pl.BlockSpec((br, DEGW), lambda i: (i, 0)),
        ],
        out_specs=[
            pl.BlockSpec((br, hd), lambda i: (i, 0)),
            pl.BlockSpec((br, 1), lambda i: (i, 0)),
        ],
        out_shape=[
            jax.ShapeDtypeStruct((n, hd), jnp.float32),
            jax.ShapeDtypeStruct((n, 1), jnp.float32),
        ],
    )(x, w1x, c_row, deg0, deg1)


def _tc_mid(acc0, acc1, dinv, b_row, w_next, br=1000):
    n, hd = acc0.shape
    grid = n // br

    def body(a0_ref, a1_ref, dinv_ref, b_ref, w_ref, hp_ref):
        dinv = dinv_ref[...]
        pre = (a0_ref[...] + a1_ref[...]) * dinv + b_ref[...]
        act = jnp.where(pre >= 0, pre, NEG_SLOPE * pre)
        h = jnp.dot(act, w_ref[...], preferred_element_type=jnp.float32)
        hp_ref[...] = h * dinv

    return pl.pallas_call(
        body,
        grid=(grid,),
        in_specs=[
            pl.BlockSpec((br, hd), lambda i: (i, 0)),
            pl.BlockSpec((br, hd), lambda i: (i, 0)),
            pl.BlockSpec((br, 1), lambda i: (i, 0)),
            pl.BlockSpec((1, hd), lambda i: (0, 0)),
            pl.BlockSpec((hd, hd), lambda i: (0, 0)),
        ],
        out_specs=pl.BlockSpec((br, hd), lambda i: (i, 0)),
        out_shape=jax.ShapeDtypeStruct((n, hd), jnp.float32),
    )(acc0, acc1, dinv, b_row, w_next)


def _tc_last(acc0, acc1, dinv, b_row, wlin_t, blin2d, br=1000):
    n, hd = acc0.shape
    grid = n // br

    def body(a0_ref, a1_ref, dinv_ref, b_ref, wl_ref, bl_ref, sum_ref, out_ref):
        i = pl.program_id(0)

        @pl.when(i == 0)
        def _():
            sum_ref[...] = jnp.zeros_like(sum_ref)

        pre = (a0_ref[...] + a1_ref[...]) * dinv_ref[...] + b_ref[...]
        act = jnp.where(pre >= 0, pre, NEG_SLOPE * pre)
        sum_ref[...] += jnp.sum(act, axis=0, keepdims=True)

        @pl.when(i == grid - 1)
        def _():
            m = sum_ref[...] * (1.0 / n)
            out_ref[...] = jnp.dot(m, wl_ref[...],
                                   preferred_element_type=jnp.float32) + bl_ref[...]

    _, out = pl.pallas_call(
        body,
        grid=(grid,),
        in_specs=[
            pl.BlockSpec((br, hd), lambda i: (i, 0)),
            pl.BlockSpec((br, hd), lambda i: (i, 0)),
            pl.BlockSpec((br, 1), lambda i: (i, 0)),
            pl.BlockSpec((1, hd), lambda i: (0, 0)),
            pl.BlockSpec((hd, 1), lambda i: (0, 0)),
            pl.BlockSpec((1, 1), lambda i: (0, 0)),
        ],
        out_specs=[
            pl.BlockSpec((1, hd), lambda i: (0, 0)),
            pl.BlockSpec((1, 1), lambda i: (0, 0)),
        ],
        out_shape=[
            jax.ShapeDtypeStruct((1, hd), jnp.float32),
            jax.ShapeDtypeStruct((1, 1), jnp.float32),
        ],
    )(acc0, acc1, dinv, b_row, wlin_t, blin2d)
    return out


def kernel(x, edge_index, u, w, W1, b1, W2, b2, W3, b3, W4, b4, W5, b5, Wlin, blin):
    n, d = x.shape
    hd = W1.shape[1]
    e = edge_index.shape[1]
    assert n % NS == 0 and e % (NC * NS * B) == 0

    src2d = edge_index[0].reshape(-1, B)
    dst2d = edge_index[1].reshape(-1, B)
    zeros_nd = jnp.zeros((n, hd), jnp.float32)
    zeros16 = jnp.zeros((n, DEGW), jnp.float32)
    ones16 = jnp.ones((B, DEGW), jnp.float32)

    degp = _degree_partials(dst2d, zeros16, ones16, n)

    u_ = jnp.asarray(u, jnp.float32)
    w_ = jnp.asarray(w, jnp.float32)
    c_row = (u_ * W1[d] + w_ * W1[d + 1]).reshape(1, hd)
    hp, dinv = _tc_first(x, W1[:d], c_row, degp[0], degp[1])

    for w_next, b_cur in ((W2, b1), (W3, b2), (W4, b3), (W5, b4)):
        acc = _sc_spmm(hp, zeros_nd, src2d, dst2d, n, hd)
        hp = _tc_mid(acc[0], acc[1], dinv, b_cur.reshape(1, hd), w_next)

    acc = _sc_spmm(hp, zeros_nd, src2d, dst2d, n, hd)
    out = _tc_last(acc[0], acc[1], dinv, b5.reshape(1, hd), Wlin.T,
                   blin.reshape(1, 1))
    return out.reshape(1)


# speedup vs baseline: 19.1777x; 2.1449x over previous
import functools

import jax
import jax.numpy as jnp
from jax import lax
from jax.experimental import pallas as pl
from jax.experimental.pallas import tpu as pltpu
from jax.experimental.pallas import tpu_sc as plsc

NC = 2
NS = 16
B = 125
DEGW = 16
NEG_SLOPE = 0.01


def _sc_mesh():
    return plsc.VectorSubcoreMesh(core_axis_name="c", subcore_axis_name="s")


_SC_PARAMS = pltpu.CompilerParams(use_tc_tiling_on_sc=False)


def _degree_partials(ec3, zeros16, ones16, n):
    nchunk, _, b = ec3.shape
    nb = nchunk // (NC * NS)
    rpt = n // NS

    @functools.partial(
        pl.kernel,
        out_type=jax.ShapeDtypeStruct((NC, n, DEGW), jnp.float32),
        mesh=_sc_mesh(),
        compiler_params=_SC_PARAMS,
        scratch_types=[
            pltpu.VMEM((2, b), jnp.int32),
            pltpu.VMEM((b, DEGW), jnp.float32),
            pltpu.VMEM_SHARED((n, DEGW), jnp.float32),
        ],
    )
    def deg_kernel(ec_hbm, zeros_hbm, ones_hbm, out_hbm, idx_v, ones_v, acc_sh):
        cid = lax.axis_index("c")
        sid = lax.axis_index("s")
        r0 = sid * rpt
        pltpu.sync_copy(zeros_hbm.at[pl.ds(r0, rpt)], acc_sh.at[pl.ds(r0, rpt)])
        pltpu.sync_copy(ones_hbm, ones_v)
        plsc.subcore_barrier()
        wid = cid * NS + sid

        def body(k, carry):
            pltpu.sync_copy(ec_hbm.at[wid * nb + k], idx_v)
            pltpu.sync_copy(ones_v, acc_sh.at[idx_v.at[1]], add=True)
            return carry

        lax.fori_loop(0, nb, body, 0)
        plsc.subcore_barrier()
        pltpu.sync_copy(acc_sh.at[pl.ds(r0, rpt)], out_hbm.at[cid, pl.ds(r0, rpt)])

    return deg_kernel(ec3, zeros16, ones16)


def _sc_spmm(hp, zeros, ec3, n, d):
    nchunk, _, b = ec3.shape
    nb = nchunk // (NC * NS)
    rpt = n // NS
    assert nb % 2 == 0

    @functools.partial(
        pl.kernel,
        out_type=jax.ShapeDtypeStruct((NC, n, d), jnp.float32),
        mesh=_sc_mesh(),
        compiler_params=_SC_PARAMS,
        scratch_types=[
            pltpu.VMEM((2, b), jnp.int32),
            pltpu.VMEM((2, b), jnp.int32),
            pltpu.VMEM((b, d), jnp.float32),
            pltpu.VMEM((b, d), jnp.float32),
            pltpu.VMEM_SHARED((n, d), jnp.float32),
            pltpu.SemaphoreType.DMA,
            pltpu.SemaphoreType.DMA,
            pltpu.SemaphoreType.DMA,
            pltpu.SemaphoreType.DMA,
        ],
    )
    def spmm_kernel(hp_hbm, zeros_hbm, ec_hbm, out_hbm,
                    idx0, idx1, rows0, rows1, acc_sh, si0, si1, sg0, sg1):
        cid = lax.axis_index("c")
        sid = lax.axis_index("s")
        r0 = sid * rpt

        @pl.when(cid == 0)
        def _():
            pltpu.sync_copy(hp_hbm.at[pl.ds(r0, rpt)], acc_sh.at[pl.ds(r0, rpt)])

        @pl.when(cid != 0)
        def _():
            pltpu.sync_copy(zeros_hbm.at[pl.ds(r0, rpt)], acc_sh.at[pl.ds(r0, rpt)])

        plsc.subcore_barrier()
        wid = cid * NS + sid
        base = wid * nb
        idx = (idx0, idx1)
        rows = (rows0, rows1)
        si = (si0, si1)
        sg = (sg0, sg1)

        pltpu.async_copy(ec_hbm.at[base], idx0, si0)
        pltpu.async_copy(ec_hbm.at[base + 1], idx1, si1)
        pltpu.make_async_copy(ec_hbm.at[base], idx0, si0).wait()
        pltpu.async_copy(hp_hbm.at[idx0.at[0]], rows0, sg0)

        def step(k, p, q):
            @pl.when(k + 1 < nb)
            def _():
                pltpu.make_async_copy(ec_hbm.at[base + k + 1], idx[q], si[q]).wait()
                pltpu.async_copy(hp_hbm.at[idx[q].at[0]], rows[q], sg[q])

            pltpu.make_async_copy(hp_hbm.at[idx[p].at[0]], rows[p], sg[p]).wait()
            pltpu.sync_copy(rows[p], acc_sh.at[idx[p].at[1]], add=True)

            @pl.when(k + 2 < nb)
            def _():
                pltpu.async_copy(ec_hbm.at[base + k + 2], idx[p], si[p])

        def body(j, carry):
            step(2 * j, 0, 1)
            step(2 * j + 1, 1, 0)
            return carry

        lax.fori_loop(0, nb // 2, body, 0)
        plsc.subcore_barrier()
        pltpu.sync_copy(acc_sh.at[pl.ds(r0, rpt)], out_hbm.at[cid, pl.ds(r0, rpt)])

    return spmm_kernel(hp, zeros, ec3)


def _tc_first(x, w1x, c_row, deg0, deg1, br=1000):
    n, d = x.shape
    hd = w1x.shape[1]
    grid = n // br

    def body(x_ref, w_ref, c_ref, d0_ref, d1_ref, hp_ref, dinv_ref):
        deg = d0_ref[:, 0:1] + d1_ref[:, 0:1] + 1.0
        dinv = lax.rsqrt(deg)
        h = jnp.dot(x_ref[...], w_ref[...],
                    preferred_element_type=jnp.float32) + c_ref[...]
        hp_ref[...] = h * dinv
        dinv_ref[...] = dinv

    return pl.pallas_call(
        body,
        grid=(grid,),
        in_specs=[
            pl.BlockSpec((br, d), lambda i: (i, 0)),
            pl.BlockSpec((d, hd), lambda i: (0, 0)),
            pl.BlockSpec((1, hd), lambda i: (0, 0)),
            pl.BlockSpec((br, DEGW), lambda i: (i, 0)),
            pl.BlockSpec((br, DEGW), lambda i: (i, 0)),
        ],
        out_specs=[
            pl.BlockSpec((br, hd), lambda i: (i, 0)),
            pl.BlockSpec((br, 1), lambda i: (i, 0)),
        ],
        out_shape=[
            jax.ShapeDtypeStruct((n, hd), jnp.float32),
            jax.ShapeDtypeStruct((n, 1), jnp.float32),
        ],
    )(x, w1x, c_row, deg0, deg1)


def _tc_mid(acc0, acc1, dinv, b_row, w_next, br=1000):
    n, hd = acc0.shape
    grid = n // br

    def body(a0_ref, a1_ref, dinv_ref, b_ref, w_ref, hp_ref):
        dinv = dinv_ref[...]
        pre = (a0_ref[...] + a1_ref[...]) * dinv + b_ref[...]
        act = jnp.where(pre >= 0, pre, NEG_SLOPE * pre)
        h = jnp.dot(act, w_ref[...], preferred_element_type=jnp.float32)
        hp_ref[...] = h * dinv

    return pl.pallas_call(
        body,
        grid=(grid,),
        in_specs=[
            pl.BlockSpec((br, hd), lambda i: (i, 0)),
            pl.BlockSpec((br, hd), lambda i: (i, 0)),
            pl.BlockSpec((br, 1), lambda i: (i, 0)),
            pl.BlockSpec((1, hd), lambda i: (0, 0)),
            pl.BlockSpec((hd, hd), lambda i: (0, 0)),
        ],
        out_specs=pl.BlockSpec((br, hd), lambda i: (i, 0)),
        out_shape=jax.ShapeDtypeStruct((n, hd), jnp.float32),
    )(acc0, acc1, dinv, b_row, w_next)


def _tc_last(acc0, acc1, dinv, b_row, wlin_t, blin2d, br=1000):
    n, hd = acc0.shape
    grid = n // br

    def body(a0_ref, a1_ref, dinv_ref, b_ref, wl_ref, bl_ref, sum_ref, out_ref):
        i = pl.program_id(0)

        @pl.when(i == 0)
        def _():
            sum_ref[...] = jnp.zeros_like(sum_ref)

        pre = (a0_ref[...] + a1_ref[...]) * dinv_ref[...] + b_ref[...]
        act = jnp.where(pre >= 0, pre, NEG_SLOPE * pre)
        sum_ref[...] += jnp.sum(act, axis=0, keepdims=True)

        @pl.when(i == grid - 1)
        def _():
            m = sum_ref[...] * (1.0 / n)
            out_ref[...] = jnp.dot(m, wl_ref[...],
                                   preferred_element_type=jnp.float32) + bl_ref[...]

    _, out = pl.pallas_call(
        body,
        grid=(grid,),
        in_specs=[
            pl.BlockSpec((br, hd), lambda i: (i, 0)),
            pl.BlockSpec((br, hd), lambda i: (i, 0)),
            pl.BlockSpec((br, 1), lambda i: (i, 0)),
            pl.BlockSpec((1, hd), lambda i: (0, 0)),
            pl.BlockSpec((hd, 1), lambda i: (0, 0)),
            pl.BlockSpec((1, 1), lambda i: (0, 0)),
        ],
        out_specs=[
            pl.BlockSpec((1, hd), lambda i: (0, 0)),
            pl.BlockSpec((1, 1), lambda i: (0, 0)),
        ],
        out_shape=[
            jax.ShapeDtypeStruct((1, hd), jnp.float32),
            jax.ShapeDtypeStruct((1, 1), jnp.float32),
        ],
    )(acc0, acc1, dinv, b_row, wlin_t, blin2d)
    return out


def kernel(x, edge_index, u, w, W1, b1, W2, b2, W3, b3, W4, b4, W5, b5, Wlin, blin):
    n, d = x.shape
    hd = W1.shape[1]
    e = edge_index.shape[1]
    assert n % NS == 0 and e % (NC * NS * B) == 0

    ec3 = edge_index.reshape(2, -1, B).transpose(1, 0, 2)
    zeros_nd = jnp.zeros((n, hd), jnp.float32)
    zeros16 = jnp.zeros((n, DEGW), jnp.float32)
    ones16 = jnp.ones((B, DEGW), jnp.float32)

    degp = _degree_partials(ec3, zeros16, ones16, n)

    u_ = jnp.asarray(u, jnp.float32)
    w_ = jnp.asarray(w, jnp.float32)
    c_row = (u_ * W1[d] + w_ * W1[d + 1]).reshape(1, hd)
    hp, dinv = _tc_first(x, W1[:d], c_row, degp[0], degp[1])

    for w_next, b_cur in ((W2, b1), (W3, b2), (W4, b3), (W5, b4)):
        acc = _sc_spmm(hp, zeros_nd, ec3, n, hd)
        hp = _tc_mid(acc[0], acc[1], dinv, b_cur.reshape(1, hd), w_next)

    acc = _sc_spmm(hp, zeros_nd, ec3, n, hd)
    out = _tc_last(acc[0], acc[1], dinv, b5.reshape(1, hd), Wlin.T,
                   blin.reshape(1, 1))
    return out.reshape(1)
